# baseline (speedup 1.0000x reference)
import functools

import jax
import jax.numpy as jnp
from jax.experimental import pallas as pl
from jax.experimental.pallas import tpu as pltpu

_B = 2
_RADIUS2 = 1.0
_NSAMPLES = (128, 32, 16)
_OFFSET = (0.0, -40.0, -3.0)
_LR_VS = (0.4, 0.4, 1.0)
_HR_VS = ((0.05, 0.05, 0.1), (0.1, 0.1, 0.2), (0.2, 0.2, 0.4))

_IB = 200
_K = 1024
_KSUB = 128
_NEG = -1e30
_FAR = 1e4


def _xyz8(coords, vs):
    vs_a = jnp.array(vs, jnp.float32)
    off = jnp.array(_OFFSET, jnp.float32)
    c = jnp.stack([coords[:, 3], coords[:, 2], coords[:, 1]], axis=1)
    xyz = c.astype(jnp.float32) * vs_a[None, :] + off[None, :] + 0.5 * vs_a[None, :]
    return jnp.pad(xyz, ((0, 0), (0, 5)))


def _pt_kernel(xyzt_ref, featt_ref, wxt_ref, wft_ref, bt_ref, out_ref):
    out_ref[...] = (
        jnp.dot(wxt_ref[...], xyzt_ref[...], preferred_element_type=jnp.float32)
        + jnp.dot(wft_ref[...], featt_ref[...], preferred_element_type=jnp.float32)
        + bt_ref[...]
    )


def _compute_pt(xyzt, featt, wxt, wft, bt):
    n = featt.shape[1]
    return pl.pallas_call(
        _pt_kernel,
        out_shape=jax.ShapeDtypeStruct((32, n), jnp.float32),
    )(xyzt, featt, wxt, wft, bt)


def _cumsum_lanes(x):
    n = x.shape[1]
    d = 1
    while d < n:
        x = x + jnp.pad(x[:, :-d], ((0, 0), (d, 0)))
        d *= 2
    return x


def _level_kernel(nsample, nj, lr_ref, hr_ref, pt_ref, wx_ref, p0_ref,
                  out_ref, rmax_ref, cnt_ref, af_ref, m_ref):
    j = pl.program_id(1)

    @pl.when(j == 0)
    def _init():
        rmax_ref[...] = jnp.full_like(rmax_ref, _NEG)
        cnt_ref[...] = jnp.zeros_like(cnt_ref)

    lr = lr_ref[...]
    hr = hr_ref[...]
    d2 = (
        jnp.sum(lr * lr, axis=1, keepdims=True)
        + jnp.sum(hr * hr, axis=1)[None, :]
        - 2.0 * jax.lax.dot_general(
            lr, hr, (((1,), (1,)), ((), ())),
            preferred_element_type=jnp.float32)
    )
    mask = d2 < _RADIUS2
    maskf = mask.astype(jnp.float32)
    tot = jnp.sum(maskf, axis=1, keepdims=True)
    cnt = cnt_ref[...]
    need_cap = jnp.max(cnt + tot) > float(nsample)

    @pl.when(need_cap)
    def _capped():
        prefix = _cumsum_lanes(maskf)
        sel = mask & ((cnt + prefix) <= float(nsample))
        af_ref[...] = sel.astype(jnp.float32)

    @pl.when(jnp.logical_not(need_cap))
    def _uncapped():
        af_ref[...] = maskf

    cnt_ref[...] = cnt + tot

    col = jax.lax.broadcasted_iota(jnp.int32, (1, 32), 1)
    for s in range(_K // _KSUB):
        af_s = af_ref[:, s * _KSUB:(s + 1) * _KSUB]
        pt_s = pt_ref[:, s * _KSUB:(s + 1) * _KSUB]
        tot_s = jnp.sum(af_s, axis=1, keepdims=True)
        sv = jax.lax.dot_general(af_s, pt_s, (((1,), (1,)), ((), ())),
                                 preferred_element_type=jnp.float32)
        m_ref[...] = jnp.where(tot_s > 0.5, sv, _NEG)

        @pl.when(jnp.max(tot_s) > 1.5)
        def _fix(af_s=af_s, pt_s=pt_s):
            chunk_m = jnp.zeros_like(m_ref)
            for c in range(32):
                prow = pt_s[c:c + 1, :]
                mc = jnp.max(jnp.where(af_s > 0.5, prow, _NEG),
                             axis=1, keepdims=True)
                chunk_m = chunk_m + mc * (col == c).astype(jnp.float32)
            m_ref[...] = chunk_m

        rmax_ref[...] = jnp.maximum(rmax_ref[...], m_ref[...])

    @pl.when(j == nj - 1)
    def _fin():
        q = jnp.dot(lr, wx_ref[...], preferred_element_type=jnp.float32)
        m = jnp.where(cnt_ref[...] > 0, rmax_ref[...], p0_ref[...])
        out_ref[...] = jnp.maximum(m - q, 0.0)


def _level_ball(lr8, hr8, pt, p0, wx, nsample):
    n_lr = lr8.shape[0]
    n_hr = hr8.shape[0]
    ni, nj = n_lr // _IB, n_hr // _K
    return pl.pallas_call(
        functools.partial(_level_kernel, nsample, nj),
        grid=(ni, nj),
        in_specs=[
            pl.BlockSpec((_IB, 8), lambda i, j: (i, 0)),
            pl.BlockSpec((_K, 8), lambda i, j: (j, 0)),
            pl.BlockSpec((32, _K), lambda i, j: (0, j)),
            pl.BlockSpec((8, 32), lambda i, j: (0, 0)),
            pl.BlockSpec((1, 32), lambda i, j: (0, 0)),
        ],
        out_specs=pl.BlockSpec((_IB, 32), lambda i, j: (i, 0)),
        out_shape=jax.ShapeDtypeStruct((n_lr, 32), jnp.float32),
        scratch_shapes=[
            pltpu.VMEM((_IB, 32), jnp.float32),
            pltpu.VMEM((_IB, 1), jnp.float32),
            pltpu.VMEM((_IB, _K), jnp.float32),
            pltpu.VMEM((_IB, 32), jnp.float32),
        ],
    )(lr8, hr8, pt, wx, p0)


def _final_kernel(f4_ref, o1_ref, o2_ref, o3_ref, w4_ref, w1_ref, w2_ref,
                  w3_ref, g_ref, be_ref, mu_ref, var_ref, out_ref):
    y = (
        jnp.dot(f4_ref[...], w4_ref[...], preferred_element_type=jnp.float32)
        + jnp.dot(o1_ref[...], w1_ref[...], preferred_element_type=jnp.float32)
        + jnp.dot(o2_ref[...], w2_ref[...], preferred_element_type=jnp.float32)
        + jnp.dot(o3_ref[...], w3_ref[...], preferred_element_type=jnp.float32)
    )
    s = g_ref[...] / jnp.sqrt(var_ref[...] + 1e-3)
    out_ref[...] = jnp.maximum((y - mu_ref[...]) * s + be_ref[...], 0.0)


def kernel(feat1, feat2, feat3, feat4, coords1, coords2, coords3, coords4,
           W14, b14, W24, b24, W34, b34, W_out,
           bn_gamma, bn_beta, bn_mean, bn_var):
    lr8 = _xyz8(coords4, _LR_VS)
    hr8s = [_xyz8(coords1, _HR_VS[0]), _xyz8(coords2, _HR_VS[1]),
            _xyz8(coords3, _HR_VS[2])]
    feats = [feat1, feat2, feat3]
    Wbs = [(W14, b14), (W24, b24), (W34, b34)]
    lr_nb = feat4.shape[0] // _B

    outs = []
    for l in range(3):
        W, b = Wbs[l]
        wx = jnp.pad(W[:3], ((0, 5), (0, 0)))
        pt = _compute_pt(hr8s[l].T, feats[l].T, wx.T, W[3:].T,
                         b.reshape(32, 1))
        hr_nb = feats[l].shape[0] // _B
        npad = -(-hr_nb // _K) * _K
        obs = []
        for bi in range(_B):
            p0 = pt[:, bi * hr_nb].reshape(1, 32)
            hr_b = jnp.pad(hr8s[l][bi * hr_nb:(bi + 1) * hr_nb],
                           ((0, npad - hr_nb), (0, 0)), constant_values=_FAR)
            pt_b = jnp.pad(pt[:, bi * hr_nb:(bi + 1) * hr_nb],
                           ((0, 0), (0, npad - hr_nb)))
            obs.append(_level_ball(
                lr8[bi * lr_nb:(bi + 1) * lr_nb],
                hr_b, pt_b, p0, wx, _NSAMPLES[l]))
        outs.append(jnp.concatenate(obs, axis=0))

    return pl.pallas_call(
        _final_kernel,
        out_shape=jax.ShapeDtypeStruct((feat4.shape[0], 64), jnp.float32),
    )(feat4, outs[0], outs[1], outs[2],
      W_out[:64], W_out[64:96], W_out[96:128], W_out[128:160],
      bn_gamma.reshape(1, 64), bn_beta.reshape(1, 64),
      bn_mean.reshape(1, 64), bn_var.reshape(1, 64))


# speedup vs baseline: 2.0978x; 2.0978x over previous
import functools

import jax
import jax.numpy as jnp
from jax.experimental import pallas as pl
from jax.experimental.pallas import tpu as pltpu

_B = 2
_RADIUS2 = 1.0
_NSAMPLES = (128, 32, 16)
_OFFSET = (0.0, -40.0, -3.0)
_LR_VS = (0.4, 0.4, 1.0)
_HR_VS = ((0.05, 0.05, 0.1), (0.1, 0.1, 0.2), (0.2, 0.2, 0.4))

_IB = 2000
_K = 1024
_NEG = -1e30
_FAR = 1e4


def _xyz8(coords, vs):
    vs_a = jnp.array(vs, jnp.float32)
    off = jnp.array(_OFFSET, jnp.float32)
    c = jnp.stack([coords[:, 3], coords[:, 2], coords[:, 1]], axis=1)
    xyz = c.astype(jnp.float32) * vs_a[None, :] + off[None, :] + 0.5 * vs_a[None, :]
    return jnp.pad(xyz, ((0, 0), (0, 5)))


def _pt_kernel(xyzt_ref, featt_ref, wxt_ref, wft_ref, bt_ref, out_ref):
    out_ref[...] = (
        jnp.dot(wxt_ref[...], xyzt_ref[...], preferred_element_type=jnp.float32)
        + jnp.dot(wft_ref[...], featt_ref[...], preferred_element_type=jnp.float32)
        + bt_ref[...]
    )


def _compute_pt(xyzt, featt, wxt, wft, bt):
    n = featt.shape[1]
    return pl.pallas_call(
        _pt_kernel,
        out_shape=jax.ShapeDtypeStruct((32, n), jnp.float32),
    )(xyzt, featt, wxt, wft, bt)


def _cumsum_lanes(x):
    n = x.shape[1]
    d = 1
    while d < n:
        x = x + jnp.pad(x[:, :-d], ((0, 0), (d, 0)))
        d *= 2
    return x


def _level_kernel(nsample, nj, lr_ref, hr_ref, pt_ref, wx_ref, p0_ref,
                  out_ref, rmax_ref, cnt_ref, pen_ref):
    j = pl.program_id(1)

    @pl.when(j == 0)
    def _init():
        rmax_ref[...] = jnp.full_like(rmax_ref, _NEG)
        cnt_ref[...] = jnp.zeros_like(cnt_ref)

    lr = lr_ref[...]
    hr = hr_ref[...]
    d2 = (
        jnp.sum(lr * lr, axis=1, keepdims=True)
        + jnp.sum(hr * hr, axis=1)[None, :]
        - 2.0 * jax.lax.dot_general(
            lr, hr, (((1,), (1,)), ((), ())),
            preferred_element_type=jnp.float32)
    )
    mask = d2 < _RADIUS2
    maskf = mask.astype(jnp.float32)
    tot = jnp.sum(maskf, axis=1, keepdims=True)
    cnt = cnt_ref[...]
    need_cap = jnp.max(cnt + tot) > float(nsample)

    @pl.when(need_cap)
    def _capped():
        prefix = _cumsum_lanes(maskf)
        sel = mask & ((cnt + prefix) <= float(nsample))
        pen_ref[...] = jnp.where(sel, 0.0, _NEG)

    @pl.when(jnp.logical_not(need_cap))
    def _uncapped():
        pen_ref[...] = jnp.where(mask, 0.0, _NEG)

    cnt_ref[...] = cnt + tot

    pen = pen_ref[...]
    chunk_m = jnp.zeros_like(rmax_ref)
    col = jax.lax.broadcasted_iota(jnp.int32, (1, 32), 1)
    for c in range(32):
        prow = pt_ref[c:c + 1, :]
        mc = jnp.max(prow + pen, axis=1, keepdims=True)
        chunk_m = chunk_m + mc * (col == c).astype(jnp.float32)
    rmax_ref[...] = jnp.maximum(rmax_ref[...], chunk_m)

    @pl.when(j == nj - 1)
    def _fin():
        q = jnp.dot(lr, wx_ref[...], preferred_element_type=jnp.float32)
        m = jnp.where(cnt_ref[...] > 0, rmax_ref[...], p0_ref[...])
        out_ref[...] = jnp.maximum(m - q, 0.0)


def _level_ball(lr8, hr8, pt, p0, wx, nsample):
    n_lr = lr8.shape[0]
    n_hr = hr8.shape[0]
    ni, nj = n_lr // _IB, n_hr // _K
    return pl.pallas_call(
        functools.partial(_level_kernel, nsample, nj),
        grid=(ni, nj),
        in_specs=[
            pl.BlockSpec((_IB, 8), lambda i, j: (i, 0)),
            pl.BlockSpec((_K, 8), lambda i, j: (j, 0)),
            pl.BlockSpec((32, _K), lambda i, j: (0, j)),
            pl.BlockSpec((8, 32), lambda i, j: (0, 0)),
            pl.BlockSpec((1, 32), lambda i, j: (0, 0)),
        ],
        out_specs=pl.BlockSpec((_IB, 32), lambda i, j: (i, 0)),
        out_shape=jax.ShapeDtypeStruct((n_lr, 32), jnp.float32),
        scratch_shapes=[
            pltpu.VMEM((_IB, 32), jnp.float32),
            pltpu.VMEM((_IB, 1), jnp.float32),
            pltpu.VMEM((_IB, _K), jnp.float32),
        ],
    )(lr8, hr8, pt, wx, p0)


def _final_kernel(f4_ref, o1_ref, o2_ref, o3_ref, w4_ref, w1_ref, w2_ref,
                  w3_ref, g_ref, be_ref, mu_ref, var_ref, out_ref):
    y = (
        jnp.dot(f4_ref[...], w4_ref[...], preferred_element_type=jnp.float32)
        + jnp.dot(o1_ref[...], w1_ref[...], preferred_element_type=jnp.float32)
        + jnp.dot(o2_ref[...], w2_ref[...], preferred_element_type=jnp.float32)
        + jnp.dot(o3_ref[...], w3_ref[...], preferred_element_type=jnp.float32)
    )
    s = g_ref[...] / jnp.sqrt(var_ref[...] + 1e-3)
    out_ref[...] = jnp.maximum((y - mu_ref[...]) * s + be_ref[...], 0.0)


def kernel(feat1, feat2, feat3, feat4, coords1, coords2, coords3, coords4,
           W14, b14, W24, b24, W34, b34, W_out,
           bn_gamma, bn_beta, bn_mean, bn_var):
    lr8 = _xyz8(coords4, _LR_VS)
    hr8s = [_xyz8(coords1, _HR_VS[0]), _xyz8(coords2, _HR_VS[1]),
            _xyz8(coords3, _HR_VS[2])]
    feats = [feat1, feat2, feat3]
    Wbs = [(W14, b14), (W24, b24), (W34, b34)]
    lr_nb = feat4.shape[0] // _B

    outs = []
    for l in range(3):
        W, b = Wbs[l]
        wx = jnp.pad(W[:3], ((0, 5), (0, 0)))
        pt = _compute_pt(hr8s[l].T, feats[l].T, wx.T, W[3:].T,
                         b.reshape(32, 1))
        hr_nb = feats[l].shape[0] // _B
        npad = -(-hr_nb // _K) * _K
        obs = []
        for bi in range(_B):
            p0 = pt[:, bi * hr_nb].reshape(1, 32)
            hr_b = jnp.pad(hr8s[l][bi * hr_nb:(bi + 1) * hr_nb],
                           ((0, npad - hr_nb), (0, 0)), constant_values=_FAR)
            pt_b = jnp.pad(pt[:, bi * hr_nb:(bi + 1) * hr_nb],
                           ((0, 0), (0, npad - hr_nb)))
            obs.append(_level_ball(
                lr8[bi * lr_nb:(bi + 1) * lr_nb],
                hr_b, pt_b, p0, wx, _NSAMPLES[l]))
        outs.append(jnp.concatenate(obs, axis=0))

    return pl.pallas_call(
        _final_kernel,
        out_shape=jax.ShapeDtypeStruct((feat4.shape[0], 64), jnp.float32),
    )(feat4, outs[0], outs[1], outs[2],
      W_out[:64], W_out[64:96], W_out[96:128], W_out[128:160],
      bn_gamma.reshape(1, 64), bn_beta.reshape(1, 64),
      bn_mean.reshape(1, 64), bn_var.reshape(1, 64))


# speedup vs baseline: 2.8337x; 1.3508x over previous
import functools

import jax
import jax.numpy as jnp
from jax.experimental import pallas as pl
from jax.experimental.pallas import tpu as pltpu

_B = 2
_RADIUS2 = 1.0
_NSAMPLES = (128, 32, 16)
_OFFSET = (0.0, -40.0, -3.0)
_LR_VS = (0.4, 0.4, 1.0)
_HR_VS = ((0.05, 0.05, 0.1), (0.1, 0.1, 0.2), (0.2, 0.2, 0.4))

_IB = 1000
_K = 1024
_NEG = -1e30
_FAR = 1e4


def _xyz8(coords, vs):
    vs_a = jnp.array(vs, jnp.float32)
    off = jnp.array(_OFFSET, jnp.float32)
    c = jnp.stack([coords[:, 3], coords[:, 2], coords[:, 1]], axis=1)
    xyz = c.astype(jnp.float32) * vs_a[None, :] + off[None, :] + 0.5 * vs_a[None, :]
    return jnp.pad(xyz, ((0, 0), (0, 5)))


def _pt_kernel(xyzt_ref, featt_ref, wxt_ref, wft_ref, bt_ref, out_ref):
    out_ref[...] = (
        jnp.dot(wxt_ref[...], xyzt_ref[...], preferred_element_type=jnp.float32)
        + jnp.dot(wft_ref[...], featt_ref[...], preferred_element_type=jnp.float32)
        + bt_ref[...]
    )


def _compute_pt(xyzt, featt, wxt, wft, bt):
    n = featt.shape[1]
    return pl.pallas_call(
        _pt_kernel,
        out_shape=jax.ShapeDtypeStruct((32, n), jnp.float32),
    )(xyzt, featt, wxt, wft, bt)


def _cumsum_lanes(x):
    n = x.shape[1]
    d = 1
    while d < n:
        x = x + jnp.pad(x[:, :-d], ((0, 0), (d, 0)))
        d *= 2
    return x


def _level_kernel(nsample, nj, lr_ref, hr_ref, pt_ref, wx_ref, p0_ref,
                  out_ref, rmax_ref, cnt_ref, pen_ref):
    j = pl.program_id(1)

    @pl.when(j == 0)
    def _init():
        rmax_ref[...] = jnp.full_like(rmax_ref, _NEG)
        cnt_ref[...] = jnp.zeros_like(cnt_ref)

    lr = lr_ref[...]
    hr = hr_ref[...]
    d2 = (
        jnp.sum(lr * lr, axis=1, keepdims=True)
        + jnp.sum(hr * hr, axis=1)[None, :]
        - 2.0 * jax.lax.dot_general(
            lr, hr, (((1,), (1,)), ((), ())),
            preferred_element_type=jnp.float32)
    )
    mask = d2 < _RADIUS2
    maskf = mask.astype(jnp.float32)
    tot = jnp.sum(maskf, axis=1, keepdims=True)
    cnt = cnt_ref[...]
    need_cap = jnp.max(cnt + tot) > float(nsample)

    @pl.when(need_cap)
    def _capped():
        prefix = _cumsum_lanes(maskf)
        sel = mask & ((cnt + prefix) <= float(nsample))
        pen_ref[...] = jnp.where(sel, 0.0, _NEG)

    @pl.when(jnp.logical_not(need_cap))
    def _uncapped():
        pen_ref[...] = jnp.where(mask, 0.0, _NEG)

    cnt_ref[...] = cnt + tot

    pen = pen_ref[...]
    chunk_m = jnp.zeros_like(rmax_ref)
    col = jax.lax.broadcasted_iota(jnp.int32, (1, 32), 1)
    for c in range(32):
        prow = pt_ref[c:c + 1, :]
        mc = jnp.max(prow + pen, axis=1, keepdims=True)
        chunk_m = chunk_m + mc * (col == c).astype(jnp.float32)
    rmax_ref[...] = jnp.maximum(rmax_ref[...], chunk_m)

    @pl.when(j == nj - 1)
    def _fin():
        q = jnp.dot(lr, wx_ref[...], preferred_element_type=jnp.float32)
        m = jnp.where(cnt_ref[...] > 0, rmax_ref[...], p0_ref[...])
        out_ref[...] = jnp.maximum(m - q, 0.0)


def _level_ball(lr8, hr8, pt, p0, wx, nsample):
    n_lr = lr8.shape[0]
    n_hr = hr8.shape[0]
    ni, nj = n_lr // _IB, n_hr // _K
    return pl.pallas_call(
        functools.partial(_level_kernel, nsample, nj),
        grid=(ni, nj),
        in_specs=[
            pl.BlockSpec((_IB, 8), lambda i, j: (i, 0)),
            pl.BlockSpec((_K, 8), lambda i, j: (j, 0)),
            pl.BlockSpec((32, _K), lambda i, j: (0, j)),
            pl.BlockSpec((8, 32), lambda i, j: (0, 0)),
            pl.BlockSpec((1, 32), lambda i, j: (0, 0)),
        ],
        out_specs=pl.BlockSpec((_IB, 32), lambda i, j: (i, 0)),
        out_shape=jax.ShapeDtypeStruct((n_lr, 32), jnp.float32),
        scratch_shapes=[
            pltpu.VMEM((_IB, 32), jnp.float32),
            pltpu.VMEM((_IB, 1), jnp.float32),
            pltpu.VMEM((_IB, _K), jnp.float32),
        ],
    )(lr8, hr8, pt, wx, p0)


def _final_kernel(f4_ref, o1_ref, o2_ref, o3_ref, w4_ref, w1_ref, w2_ref,
                  w3_ref, g_ref, be_ref, mu_ref, var_ref, out_ref):
    y = (
        jnp.dot(f4_ref[...], w4_ref[...], preferred_element_type=jnp.float32)
        + jnp.dot(o1_ref[...], w1_ref[...], preferred_element_type=jnp.float32)
        + jnp.dot(o2_ref[...], w2_ref[...], preferred_element_type=jnp.float32)
        + jnp.dot(o3_ref[...], w3_ref[...], preferred_element_type=jnp.float32)
    )
    s = g_ref[...] / jnp.sqrt(var_ref[...] + 1e-3)
    out_ref[...] = jnp.maximum((y - mu_ref[...]) * s + be_ref[...], 0.0)


def kernel(feat1, feat2, feat3, feat4, coords1, coords2, coords3, coords4,
           W14, b14, W24, b24, W34, b34, W_out,
           bn_gamma, bn_beta, bn_mean, bn_var):
    lr8 = _xyz8(coords4, _LR_VS)
    hr8s = [_xyz8(coords1, _HR_VS[0]), _xyz8(coords2, _HR_VS[1]),
            _xyz8(coords3, _HR_VS[2])]
    feats = [feat1, feat2, feat3]
    Wbs = [(W14, b14), (W24, b24), (W34, b34)]
    lr_nb = feat4.shape[0] // _B

    outs = []
    for l in range(3):
        W, b = Wbs[l]
        wx = jnp.pad(W[:3], ((0, 5), (0, 0)))
        pt = _compute_pt(hr8s[l].T, feats[l].T, wx.T, W[3:].T,
                         b.reshape(32, 1))
        hr_nb = feats[l].shape[0] // _B
        npad = -(-hr_nb // _K) * _K
        obs = []
        for bi in range(_B):
            p0 = pt[:, bi * hr_nb].reshape(1, 32)
            hr_b = jnp.pad(hr8s[l][bi * hr_nb:(bi + 1) * hr_nb],
                           ((0, npad - hr_nb), (0, 0)), constant_values=_FAR)
            pt_b = jnp.pad(pt[:, bi * hr_nb:(bi + 1) * hr_nb],
                           ((0, 0), (0, npad - hr_nb)))
            obs.append(_level_ball(
                lr8[bi * lr_nb:(bi + 1) * lr_nb],
                hr_b, pt_b, p0, wx, _NSAMPLES[l]))
        outs.append(jnp.concatenate(obs, axis=0))

    return pl.pallas_call(
        _final_kernel,
        out_shape=jax.ShapeDtypeStruct((feat4.shape[0], 64), jnp.float32),
    )(feat4, outs[0], outs[1], outs[2],
      W_out[:64], W_out[64:96], W_out[96:128], W_out[128:160],
      bn_gamma.reshape(1, 64), bn_beta.reshape(1, 64),
      bn_mean.reshape(1, 64), bn_var.reshape(1, 64))
